# baseline (speedup 1.0000x reference)
import jax
import jax.numpy as jnp
from jax import lax
from jax.experimental import pallas as pl
from jax.experimental.pallas import tpu as pltpu
from jax.experimental.pallas import tpu_sc as plsc

_NC = 2
_NS = 16
_NW = _NC * _NS


def _sc_merged(ent_emb, usr_emb, rel_emb, adjE8, adjR8, u32, v32):
    mesh = plsc.VectorSubcoreMesh(core_axis_name="c", subcore_axis_name="s")

    def body(ent, usr, rel, adjE, adjR, u_in, v_in,
             u_out, ev0_out, ev1_out, agg_out, w0_out,
             v_v, u_v, sidx, ubuf, e0buf,
             abufA0, abufA1, abufB0, abufB1,
             e1_v, r0_v, rel_v, sco_v, idx64a, idx64b,
             e2_v, r1_v, buf0, buf1, stage, w0_v,
             semA0, semA1, semB0, semB1, sem0, sem1):
        wid = lax.axis_index("s") * _NC + lax.axis_index("c")
        lane = lax.iota(jnp.int32, 16)
        pltpu.sync_copy(v_in.at[pl.ds(wid, 1)], v_v)
        pltpu.sync_copy(u_in.at[pl.ds(wid, 1)], u_v)
        pltpu.sync_copy(rel, rel_v)

        for g in range(2):
            t = v_v[0, pl.ds(g * 16, 16)]
            sidx[pl.ds(g * 16, 16)] = lax.shift_right_logical(t, 3)
        pltpu.async_copy(adjE.at[sidx], abufA0.at[pl.ds(0, 32)], semA0)
        pltpu.async_copy(adjR.at[sidx], abufB0.at[pl.ds(0, 32)], semB0)
        pltpu.async_copy(usr.at[u_v.at[0]], ubuf, sem0)
        pltpu.async_copy(ent.at[v_v.at[0]], e0buf, sem1)
        pltpu.make_async_copy(adjE.at[sidx], abufA0.at[pl.ds(0, 32)],
                              semA0).wait()
        pltpu.make_async_copy(adjR.at[sidx], abufB0.at[pl.ds(0, 32)],
                              semB0).wait()
        pltpu.make_async_copy(usr.at[u_v.at[0]], ubuf, sem0).wait()
        pltpu.make_async_copy(ent.at[v_v.at[0]], e0buf, sem1).wait()
        pltpu.sync_copy(ubuf, u_out.at[pl.ds(wid * 32, 32)])
        pltpu.sync_copy(e0buf, ev0_out.at[pl.ds(wid * 32, 32)])
        for gg in range(2):
            tv = v_v[0, pl.ds(gg * 16, 16)]
            kbase = (tv & 7) * 16
            rows = lane + gg * 16
            for j in range(16):
                flat = rows * 16 + j
                plsc.store_scatter(
                    e1_v, [lax.shift_right_logical(flat, 7), flat & 127],
                    plsc.load_gather(abufA0, [rows, kbase + j]))
                plsc.store_scatter(
                    r0_v, [lax.shift_right_logical(flat, 7), flat & 127],
                    plsc.load_gather(abufB0, [rows, kbase + j]))

        idxs = (idx64a, idx64b)
        abufsA = (abufA0, abufA1)
        abufsB = (abufB0, abufB1)
        semsA = (semA0, semA1)
        semsB = (semB0, semB1)
        bufs = (buf0, buf1)
        sems = (sem0, sem1)

        def shift_chunk(c, dst):
            for g in range(2):
                off = pl.multiple_of((c & 3) * 32 + g * 16, 16)
                t = e1_v[lax.shift_right_logical(c, 2), pl.ds(off, 16)]
                dst[pl.ds(g * 16, 16)] = lax.shift_right_logical(t, 3)

        def issueC(c, sub):
            pltpu.async_copy(adjE.at[idxs[sub]], abufsA[sub], semsA[sub])
            pltpu.async_copy(adjR.at[idxs[sub]], abufsB[sub], semsB[sub])

        def ev1_issue(c, sub):
            pltpu.async_copy(ent.at[e1_v.at[c]], bufs[sub], sems[sub])

        shift_chunk(0, idx64a)
        issueC(0, 0)
        shift_chunk(1, idx64b)
        issueC(1, 1)
        ev1_issue(0, 0)
        ev1_issue(1, 1)

        @plsc.parallel_loop(0, 32, unroll=1)
        def score_item(i):
            urow = [ubuf[i, pl.ds(jj * 16, 16)] for jj in range(8)]
            svec = [jnp.zeros((16,), jnp.float32) for _ in range(2)]
            for r in range(32):
                ps = [urow[jj] * rel_v[r, pl.ds(jj * 16, 16)]
                      for jj in range(8)]
                for step in (4, 2, 1):
                    ps = [ps[t] + ps[t + step] for t in range(step)]
                s = jnp.sum(ps[0])
                svec[r >> 4] = jnp.where(lane == (r & 15), s, svec[r >> 4])
            base = pl.multiple_of((i & 3) * 32, 32)
            sco_v[i >> 2, pl.ds(base, 16)] = svec[0]
            sco_v[i >> 2, pl.ds(pl.multiple_of(base + 16, 16), 16)] = svec[1]

        def cpair(p, _):
            for sub in range(2):
                c = 2 * p + sub
                pltpu.make_async_copy(adjE.at[idxs[sub]], abufsA[sub],
                                      semsA[sub]).wait()
                pltpu.make_async_copy(adjR.at[idxs[sub]], abufsB[sub],
                                      semsB[sub]).wait()
                crow = lax.shift_right_logical(c, 2)
                cofs = (c & 3) * 32
                for gg in range(2):
                    off = pl.multiple_of(cofs + gg * 16, 16)
                    tv = e1_v[crow, pl.ds(off, 16)]
                    kbase = (tv & 7) * 16
                    rows = lane + gg * 16
                    for j in range(16):
                        flat = rows * 16 + j
                        er = c * 4 + lax.shift_right_logical(flat, 7)
                        plsc.store_scatter(
                            e2_v, [er, flat & 127],
                            plsc.load_gather(abufsA[sub],
                                             [rows, kbase + j]))
                        plsc.store_scatter(
                            r1_v, [er, flat & 127],
                            plsc.load_gather(abufsB[sub],
                                             [rows, kbase + j]))
                nxt = jnp.minimum(c + 2, 15)
                shift_chunk(nxt, idxs[sub])
                issueC(nxt, sub)
            return _
        lax.fori_loop(0, 8, cpair, None)
        for sub in range(2):
            pltpu.make_async_copy(adjE.at[idxs[sub]], abufsA[sub],
                                  semsA[sub]).wait()
            pltpu.make_async_copy(adjR.at[idxs[sub]], abufsB[sub],
                                  semsB[sub]).wait()

        def ev1_wait(sub):
            pltpu.make_async_copy(ent.at[e1_v.at[0]], bufs[sub],
                                  sems[sub]).wait()

        for c in range(4):
            sub = c & 1
            ev1_wait(sub)
            pltpu.sync_copy(bufs[sub],
                            ev1_out.at[pl.ds(wid * 512 + c * 128, 128)])
            if c + 2 < 4:
                ev1_issue(c + 2, sub)

        def softmax16(svals):
            m = jnp.max(svals)
            es = jnp.exp(svals - m)
            return es / jnp.sum(es)

        @plsc.parallel_loop(0, 32, unroll=4)
        def w0_item(i):
            off = pl.multiple_of((i & 7) * 16, 16)
            rv = r0_v[i >> 3, pl.ds(off, 16)]
            srow = jnp.broadcast_to(i >> 2, (16,))
            svals = plsc.load_gather(sco_v, [srow, (i & 3) * 32 + rv])
            w0_v[i >> 3, pl.ds(off, 16)] = softmax16(svals)
        pltpu.sync_copy(w0_v, w0_out.at[pl.ds(wid * 4, 4)])

        def issueD(c, sub):
            pltpu.async_copy(ent.at[e2_v.at[jnp.minimum(c, 63)]],
                             bufs[sub], sems[sub])

        def processD(c, cl, sub):
            buf = bufs[sub]
            item = c >> 1

            @plsc.parallel_loop(0, 8, unroll=4)
            def group(g):
                coff = pl.multiple_of(g * 16, 16)
                rv = r1_v[c, pl.ds(coff, 16)]
                srow = jnp.broadcast_to(item >> 2, (16,))
                svals = plsc.load_gather(sco_v, [srow, (item & 3) * 32 + rv])
                w = softmax16(svals)
                srow16 = (cl >> 1) * 16 + (c & 1) * 8 + g
                wks = [jnp.broadcast_to(w[k], (16,)) for k in range(16)]
                for j in range(8):
                    parts = [
                        wks[k] * buf[g * 16 + k, pl.ds(j * 16, 16)]
                        for k in range(16)]
                    for step in (8, 4, 2, 1):
                        parts = [parts[t] + parts[t + step]
                                 for t in range(step)]
                    stage[srow16, pl.ds(j * 16, 16)] = parts[0]

        issueD(0, 0)
        issueD(1, 1)
        for h in range(2):
            def dpair(p, _, h=h):
                cl = 2 * p
                c = h * 32 + cl
                for sub in range(2):
                    pltpu.make_async_copy(ent.at[e2_v.at[0]], bufs[sub],
                                          sems[sub]).wait()
                    processD(c + sub, cl + sub, sub)
                    issueD(c + sub + 2, sub)
                return _
            lax.fori_loop(0, 16, dpair, None)
            pltpu.sync_copy(
                stage, agg_out.at[pl.ds(wid * 512 + h * 256, 256)])
        pltpu.make_async_copy(ent.at[e2_v.at[0]], bufs[0], sems[0]).wait()
        pltpu.make_async_copy(ent.at[e2_v.at[0]], bufs[1], sems[1]).wait()

    f = pl.kernel(
        body,
        out_type=(jax.ShapeDtypeStruct((1024, 128), jnp.float32),
                  jax.ShapeDtypeStruct((1024, 128), jnp.float32),
                  jax.ShapeDtypeStruct((16384, 128), jnp.float32),
                  jax.ShapeDtypeStruct((16384, 128), jnp.float32),
                  jax.ShapeDtypeStruct((128, 128), jnp.float32)),
        mesh=mesh,
        scratch_types=(
            pltpu.VMEM((1, 32), jnp.int32),
            pltpu.VMEM((1, 32), jnp.int32),
            pltpu.VMEM((32,), jnp.int32),
            pltpu.VMEM((32, 128), jnp.float32),
            pltpu.VMEM((32, 128), jnp.float32),
            pltpu.VMEM((32, 128), jnp.int32),
            pltpu.VMEM((32, 128), jnp.int32),
            pltpu.VMEM((32, 128), jnp.int32),
            pltpu.VMEM((32, 128), jnp.int32),
            pltpu.VMEM((4, 128), jnp.int32),
            pltpu.VMEM((4, 128), jnp.int32),
            pltpu.VMEM((32, 128), jnp.float32),
            pltpu.VMEM((8, 128), jnp.float32),
            pltpu.VMEM((32,), jnp.int32),
            pltpu.VMEM((32,), jnp.int32),
            pltpu.VMEM((64, 128), jnp.int32),
            pltpu.VMEM((64, 128), jnp.int32),
            pltpu.VMEM((128, 128), jnp.float32),
            pltpu.VMEM((128, 128), jnp.float32),
            pltpu.VMEM((256, 128), jnp.float32),
            pltpu.VMEM((4, 128), jnp.float32),
            pltpu.SemaphoreType.DMA,
            pltpu.SemaphoreType.DMA,
            pltpu.SemaphoreType.DMA,
            pltpu.SemaphoreType.DMA,
            pltpu.SemaphoreType.DMA,
            pltpu.SemaphoreType.DMA,
        ),
        compiler_params=pltpu.CompilerParams(needs_layout_passes=False))
    return f(ent_emb, usr_emb, rel_emb, adjE8, adjR8, u32, v32)


def _dense_body(u_ref, ev0_ref, ev1_ref, ag1_ref, w0_ref, w_ref, b_ref,
                out_ref):
    bb = u_ref.shape[0]
    U = u_ref[...]
    W = w_ref[...]
    bias = b_ref[...]
    EV0 = ev0_ref[...]
    EV1 = ev1_ref[...]
    agg1 = ag1_ref[...]
    w0 = w0_ref[...]

    h1 = jax.nn.sigmoid(
        jnp.dot((EV1 + agg1).reshape(bb * 16, 128), W,
                preferred_element_type=jnp.float32) + bias
    ).reshape(bb, 16, 128)
    agg0 = jnp.sum(w0[..., None] * EV1, axis=1)
    h0 = jax.nn.sigmoid(
        jnp.dot(EV0 + agg0, W, preferred_element_type=jnp.float32) + bias)
    agg0b = jnp.sum(w0[..., None] * h1, axis=1)
    final = jnp.tanh(
        jnp.dot(h0 + agg0b, W, preferred_element_type=jnp.float32) + bias)
    out_ref[...] = jax.nn.sigmoid(jnp.sum(U * final, axis=1))[:, None]


def _tc_dense(U, EV0, EV1, AG1, w0, W, bvec):
    B = U.shape[0]
    bb = 128
    grid = B // bb
    return pl.pallas_call(
        _dense_body,
        grid=(grid,),
        in_specs=[
            pl.BlockSpec((bb, 128), lambda i: (i, 0)),
            pl.BlockSpec((bb, 128), lambda i: (i, 0)),
            pl.BlockSpec((bb, 16, 128), lambda i: (i, 0, 0)),
            pl.BlockSpec((bb, 16, 128), lambda i: (i, 0, 0)),
            pl.BlockSpec((bb, 16), lambda i: (i, 0)),
            pl.BlockSpec((128, 128), lambda i: (0, 0)),
            pl.BlockSpec((1, 128), lambda i: (0, 0)),
        ],
        out_specs=pl.BlockSpec((bb, 1), lambda i: (i, 0)),
        out_shape=jax.ShapeDtypeStruct((B, 1), jnp.float32),
    )(U, EV0, EV1, AG1, w0, W, bvec.reshape(1, 128))


def kernel(ent_emb, usr_emb, rel_emb, W, b, adj_ent, adj_rel, u, v):
    B = u.shape[0]
    n_nb = adj_ent.shape[1]

    U, EV0, EV1, AGG1, W0 = _sc_merged(
        ent_emb, usr_emb, rel_emb,
        adj_ent.reshape(-1, 128), adj_rel.reshape(-1, 128),
        u.reshape(32, 32), v.reshape(32, 32))

    out = _tc_dense(
        U, EV0,
        EV1.reshape(B, n_nb, 128),
        AGG1.reshape(B, n_nb, 128),
        W0.reshape(B, n_nb),
        W, b)
    return out.reshape(B)


# speedup vs baseline: 1.1755x; 1.1755x over previous
import jax
import jax.numpy as jnp
from jax import lax
from jax.experimental import pallas as pl
from jax.experimental.pallas import tpu as pltpu
from jax.experimental.pallas import tpu_sc as plsc

_NC = 2
_NS = 16
_NW = _NC * _NS


def _sc_merged(ent_emb, usr_emb, rel_emb, adjE8, adjR8, u32, v32):
    mesh = plsc.VectorSubcoreMesh(core_axis_name="c", subcore_axis_name="s")

    def body(ent, usr, rel, adjE, adjR, u_in, v_in,
             u_out, ev0_out, ev1_out, agg_out, w0_out,
             v_v, u_v, sidx, ubuf, e0buf,
             abufA0, abufA1, abufB0, abufB1,
             e1_v, r0_v, rel_v, sco_v, idx64a, idx64b,
             e2_v, r1_v, buf0, buf1, stage, w0_v,
             semA0, semA1, semB0, semB1, sem0, sem1):
        wid = lax.axis_index("s") * _NC + lax.axis_index("c")
        lane = lax.iota(jnp.int32, 16)
        pltpu.sync_copy(v_in.at[pl.ds(wid, 1)], v_v)
        pltpu.sync_copy(u_in.at[pl.ds(wid, 1)], u_v)
        pltpu.sync_copy(rel, rel_v)

        for g in range(2):
            t = v_v[0, pl.ds(g * 16, 16)]
            sidx[pl.ds(g * 16, 16)] = lax.shift_right_logical(t, 3)
        pltpu.async_copy(adjE.at[sidx], abufA0.at[pl.ds(0, 32)], semA0)
        pltpu.async_copy(adjR.at[sidx], abufB0.at[pl.ds(0, 32)], semB0)
        pltpu.async_copy(usr.at[u_v.at[0]], ubuf, sem0)
        pltpu.async_copy(ent.at[v_v.at[0]], e0buf, sem1)
        pltpu.make_async_copy(adjE.at[sidx], abufA0.at[pl.ds(0, 32)],
                              semA0).wait()
        pltpu.make_async_copy(adjR.at[sidx], abufB0.at[pl.ds(0, 32)],
                              semB0).wait()
        pltpu.make_async_copy(usr.at[u_v.at[0]], ubuf, sem0).wait()
        pltpu.make_async_copy(ent.at[v_v.at[0]], e0buf, sem1).wait()
        pltpu.sync_copy(ubuf, u_out.at[pl.ds(wid * 32, 32)])
        pltpu.sync_copy(e0buf, ev0_out.at[pl.ds(wid * 32, 32)])
        for gg in range(2):
            tv = v_v[0, pl.ds(gg * 16, 16)]
            kbase = (tv & 7) * 16
            rows = lane + gg * 16
            for j in range(16):
                flat = rows * 16 + j
                plsc.store_scatter(
                    e1_v, [lax.shift_right_logical(flat, 7), flat & 127],
                    plsc.load_gather(abufA0, [rows, kbase + j]))
                plsc.store_scatter(
                    r0_v, [lax.shift_right_logical(flat, 7), flat & 127],
                    plsc.load_gather(abufB0, [rows, kbase + j]))

        idxs = (idx64a, idx64b)
        abufsA = (abufA0, abufA1)
        abufsB = (abufB0, abufB1)
        semsA = (semA0, semA1)
        semsB = (semB0, semB1)
        bufs = (buf0, buf1)
        sems = (sem0, sem1)

        def shift_chunk(c, dst):
            for g in range(2):
                off = pl.multiple_of((c & 3) * 32 + g * 16, 16)
                t = e1_v[lax.shift_right_logical(c, 2), pl.ds(off, 16)]
                dst[pl.ds(g * 16, 16)] = lax.shift_right_logical(t, 3)

        def issueC(c, sub):
            pltpu.async_copy(adjE.at[idxs[sub]], abufsA[sub], semsA[sub])
            pltpu.async_copy(adjR.at[idxs[sub]], abufsB[sub], semsB[sub])

        def ev1_issue(c, sub):
            pltpu.async_copy(ent.at[e1_v.at[c]], bufs[sub], sems[sub])

        shift_chunk(0, idx64a)
        issueC(0, 0)
        shift_chunk(1, idx64b)
        issueC(1, 1)
        ev1_issue(0, 0)
        ev1_issue(1, 1)

        @plsc.parallel_loop(0, 32, unroll=2)
        def score_item(i):
            urow = [ubuf[i, pl.ds(jj * 16, 16)] for jj in range(8)]
            svec = [jnp.zeros((16,), jnp.float32) for _ in range(2)]
            for r in range(32):
                ps = [urow[jj] * rel_v[r, pl.ds(jj * 16, 16)]
                      for jj in range(8)]
                for step in (4, 2, 1):
                    ps = [ps[t] + ps[t + step] for t in range(step)]
                s = jnp.sum(ps[0])
                svec[r >> 4] = jnp.where(lane == (r & 15), s, svec[r >> 4])
            base = pl.multiple_of((i & 3) * 32, 32)
            sco_v[i >> 2, pl.ds(base, 16)] = svec[0]
            sco_v[i >> 2, pl.ds(pl.multiple_of(base + 16, 16), 16)] = svec[1]

        def cpair(p, _):
            for sub in range(2):
                c = 2 * p + sub
                pltpu.make_async_copy(adjE.at[idxs[sub]], abufsA[sub],
                                      semsA[sub]).wait()
                pltpu.make_async_copy(adjR.at[idxs[sub]], abufsB[sub],
                                      semsB[sub]).wait()
                crow = lax.shift_right_logical(c, 2)
                cofs = (c & 3) * 32
                for gg in range(2):
                    off = pl.multiple_of(cofs + gg * 16, 16)
                    tv = e1_v[crow, pl.ds(off, 16)]
                    kbase = (tv & 7) * 16
                    rows = lane + gg * 16
                    for j in range(16):
                        flat = rows * 16 + j
                        er = c * 4 + lax.shift_right_logical(flat, 7)
                        plsc.store_scatter(
                            e2_v, [er, flat & 127],
                            plsc.load_gather(abufsA[sub],
                                             [rows, kbase + j]))
                        plsc.store_scatter(
                            r1_v, [er, flat & 127],
                            plsc.load_gather(abufsB[sub],
                                             [rows, kbase + j]))
                nxt = jnp.minimum(c + 2, 15)
                shift_chunk(nxt, idxs[sub])
                issueC(nxt, sub)
            return _
        lax.fori_loop(0, 8, cpair, None)
        for sub in range(2):
            pltpu.make_async_copy(adjE.at[idxs[sub]], abufsA[sub],
                                  semsA[sub]).wait()
            pltpu.make_async_copy(adjR.at[idxs[sub]], abufsB[sub],
                                  semsB[sub]).wait()

        def ev1_wait(sub):
            pltpu.make_async_copy(ent.at[e1_v.at[0]], bufs[sub],
                                  sems[sub]).wait()

        for c in range(4):
            sub = c & 1
            ev1_wait(sub)
            pltpu.sync_copy(bufs[sub],
                            ev1_out.at[pl.ds(wid * 512 + c * 128, 128)])
            if c + 2 < 4:
                ev1_issue(c + 2, sub)

        def softmax16(svals):
            m = jnp.max(svals)
            es = jnp.exp(svals - m)
            return es / jnp.sum(es)

        @plsc.parallel_loop(0, 32, unroll=4)
        def w0_item(i):
            off = pl.multiple_of((i & 7) * 16, 16)
            rv = r0_v[i >> 3, pl.ds(off, 16)]
            srow = jnp.broadcast_to(i >> 2, (16,))
            svals = plsc.load_gather(sco_v, [srow, (i & 3) * 32 + rv])
            w0_v[i >> 3, pl.ds(off, 16)] = softmax16(svals)
        pltpu.sync_copy(w0_v, w0_out.at[pl.ds(wid * 4, 4)])

        def issueD(c, sub):
            pltpu.async_copy(ent.at[e2_v.at[jnp.minimum(c, 63)]],
                             bufs[sub], sems[sub])

        def processD(c, cl, sub):
            buf = bufs[sub]
            item = c >> 1

            @plsc.parallel_loop(0, 8, unroll=2)
            def group(g):
                coff = pl.multiple_of(g * 16, 16)
                rv = r1_v[c, pl.ds(coff, 16)]
                srow = jnp.broadcast_to(item >> 2, (16,))
                svals = plsc.load_gather(sco_v, [srow, (item & 3) * 32 + rv])
                w = softmax16(svals)
                srow16 = (cl >> 1) * 16 + (c & 1) * 8 + g
                wks = [jnp.broadcast_to(w[k], (16,)) for k in range(16)]
                for j in range(8):
                    parts = [
                        wks[k] * buf[g * 16 + k, pl.ds(j * 16, 16)]
                        for k in range(16)]
                    for step in (8, 4, 2, 1):
                        parts = [parts[t] + parts[t + step]
                                 for t in range(step)]
                    stage[srow16, pl.ds(j * 16, 16)] = parts[0]

        issueD(0, 0)
        issueD(1, 1)
        for h in range(2):
            def dpair(p, _, h=h):
                cl = 2 * p
                c = h * 32 + cl
                for sub in range(2):
                    pltpu.make_async_copy(ent.at[e2_v.at[0]], bufs[sub],
                                          sems[sub]).wait()
                    processD(c + sub, cl + sub, sub)
                    issueD(c + sub + 2, sub)
                return _
            lax.fori_loop(0, 16, dpair, None)
            pltpu.sync_copy(
                stage, agg_out.at[pl.ds(wid * 512 + h * 256, 256)])
        pltpu.make_async_copy(ent.at[e2_v.at[0]], bufs[0], sems[0]).wait()
        pltpu.make_async_copy(ent.at[e2_v.at[0]], bufs[1], sems[1]).wait()

    f = pl.kernel(
        body,
        out_type=(jax.ShapeDtypeStruct((1024, 128), jnp.float32),
                  jax.ShapeDtypeStruct((1024, 128), jnp.float32),
                  jax.ShapeDtypeStruct((16384, 128), jnp.float32),
                  jax.ShapeDtypeStruct((16384, 128), jnp.float32),
                  jax.ShapeDtypeStruct((128, 128), jnp.float32)),
        mesh=mesh,
        scratch_types=(
            pltpu.VMEM((1, 32), jnp.int32),
            pltpu.VMEM((1, 32), jnp.int32),
            pltpu.VMEM((32,), jnp.int32),
            pltpu.VMEM((32, 128), jnp.float32),
            pltpu.VMEM((32, 128), jnp.float32),
            pltpu.VMEM((32, 128), jnp.int32),
            pltpu.VMEM((32, 128), jnp.int32),
            pltpu.VMEM((32, 128), jnp.int32),
            pltpu.VMEM((32, 128), jnp.int32),
            pltpu.VMEM((4, 128), jnp.int32),
            pltpu.VMEM((4, 128), jnp.int32),
            pltpu.VMEM((32, 128), jnp.float32),
            pltpu.VMEM((8, 128), jnp.float32),
            pltpu.VMEM((32,), jnp.int32),
            pltpu.VMEM((32,), jnp.int32),
            pltpu.VMEM((64, 128), jnp.int32),
            pltpu.VMEM((64, 128), jnp.int32),
            pltpu.VMEM((128, 128), jnp.float32),
            pltpu.VMEM((128, 128), jnp.float32),
            pltpu.VMEM((256, 128), jnp.float32),
            pltpu.VMEM((4, 128), jnp.float32),
            pltpu.SemaphoreType.DMA,
            pltpu.SemaphoreType.DMA,
            pltpu.SemaphoreType.DMA,
            pltpu.SemaphoreType.DMA,
            pltpu.SemaphoreType.DMA,
            pltpu.SemaphoreType.DMA,
        ),
        compiler_params=pltpu.CompilerParams(needs_layout_passes=False))
    return f(ent_emb, usr_emb, rel_emb, adjE8, adjR8, u32, v32)


def _dense_body(u_ref, ev0_ref, ev1_ref, ag1_ref, w0_ref, w_ref, b_ref,
                out_ref):
    bb = u_ref.shape[0]
    U = u_ref[...]
    W = w_ref[...]
    bias = b_ref[...]
    EV0 = ev0_ref[...]
    EV1 = ev1_ref[...]
    agg1 = ag1_ref[...]
    w0 = w0_ref[...]

    h1 = jax.nn.sigmoid(
        jnp.dot((EV1 + agg1).reshape(bb * 16, 128), W,
                preferred_element_type=jnp.float32) + bias
    ).reshape(bb, 16, 128)
    agg0 = jnp.sum(w0[..., None] * EV1, axis=1)
    h0 = jax.nn.sigmoid(
        jnp.dot(EV0 + agg0, W, preferred_element_type=jnp.float32) + bias)
    agg0b = jnp.sum(w0[..., None] * h1, axis=1)
    final = jnp.tanh(
        jnp.dot(h0 + agg0b, W, preferred_element_type=jnp.float32) + bias)
    out_ref[...] = jax.nn.sigmoid(jnp.sum(U * final, axis=1))[:, None]


def _tc_dense(U, EV0, EV1, AG1, w0, W, bvec):
    B = U.shape[0]
    bb = 128
    grid = B // bb
    return pl.pallas_call(
        _dense_body,
        grid=(grid,),
        in_specs=[
            pl.BlockSpec((bb, 128), lambda i: (i, 0)),
            pl.BlockSpec((bb, 128), lambda i: (i, 0)),
            pl.BlockSpec((bb, 16, 128), lambda i: (i, 0, 0)),
            pl.BlockSpec((bb, 16, 128), lambda i: (i, 0, 0)),
            pl.BlockSpec((bb, 16), lambda i: (i, 0)),
            pl.BlockSpec((128, 128), lambda i: (0, 0)),
            pl.BlockSpec((1, 128), lambda i: (0, 0)),
        ],
        out_specs=pl.BlockSpec((bb, 1), lambda i: (i, 0)),
        out_shape=jax.ShapeDtypeStruct((B, 1), jnp.float32),
    )(U, EV0, EV1, AG1, w0, W, bvec.reshape(1, 128))


def kernel(ent_emb, usr_emb, rel_emb, W, b, adj_ent, adj_rel, u, v):
    B = u.shape[0]
    n_nb = adj_ent.shape[1]

    U, EV0, EV1, AGG1, W0 = _sc_merged(
        ent_emb, usr_emb, rel_emb,
        adj_ent.reshape(-1, 128), adj_rel.reshape(-1, 128),
        u.reshape(32, 32), v.reshape(32, 32))

    out = _tc_dense(
        U, EV0,
        EV1.reshape(B, n_nb, 128),
        AGG1.reshape(B, n_nb, 128),
        W0.reshape(B, n_nb),
        W, b)
    return out.reshape(B)
